# baseline (speedup 1.0000x reference)
import dataclasses
import functools

import jax
import jax.numpy as jnp
from jax import lax
from jax.experimental import pallas as pl
from jax.experimental.pallas import tpu as pltpu
from jax.experimental.pallas import tpu_sc as plsc

N = 10000
K = 32
NP_PAD = 10240
NP_BIG = 11264
PTS = 4
EPC = PTS * K
CPI = 4
OROWS = CPI * PTS
RING = 4


def _lrelu(x):
    return jnp.maximum(x, 0.1 * x)


def _dot(a, b):
    return jnp.dot(a, b, preferred_element_type=jnp.float32,
                   precision=lax.Precision.HIGHEST)


def _perm_pack(x):
    rb, c = x.shape
    return x.reshape(rb, c // 32, 2, 16).swapaxes(2, 3).reshape(rb, c)


def _prep1_body(xt_ref, xyzt_ref, wlin1t_ref, blin1_ref, wwn1t_ref,
                wwn2t_ref, t1_ref, a2_ref):
    f1 = _lrelu(_dot(xt_ref[...], wlin1t_ref[...]) + blin1_ref[...])
    a1 = _dot(xyzt_ref[...], wwn1t_ref[...])
    t1_ref[:, :128] = _perm_pack(f1 * (1.0 / K)).astype(jnp.bfloat16)
    t1_ref[:, 128:] = _perm_pack(a1).astype(jnp.bfloat16)
    a2_ref[...] = _dot(xyzt_ref[...], wwn2t_ref[...])


def _prep2_body(x_ref, wlin2t_ref, blin2_ref, a2_ref, t2_ref):
    f2 = _lrelu(_dot(x_ref[...], wlin2t_ref[...]) + blin2_ref[...])
    t2_ref[:, :64] = f2 * (1.0 / K)
    t2_ref[:, 64:] = a2_ref[...]


def _final_body(x_ref, wfct_ref, bfc_ref, r_ref):
    r_ref[...] = _dot(x_ref[...], wfct_ref[...]) + bfc_ref[...]


def _make_sc_layer(C, ppw0, ppw1, packed):
    GP = C // 32
    G = C // 16
    ppw_max = max(ppw0, ppw1)
    rdt = jnp.int32 if packed else jnp.float32
    rcols = C if packed else 2 * C
    bdt = jnp.bfloat16 if packed else jnp.float32
    mesh = plsc.VectorSubcoreMesh(core_axis_name="c", subcore_axis_name="s")
    cp = pltpu.CompilerParams()
    if packed and "needs_layout_passes" in pltpu.CompilerParams.__dataclass_fields__:
        cp = dataclasses.replace(cp, needs_layout_passes=False)

    @functools.partial(
        pl.kernel,
        mesh=mesh,
        compiler_params=cp,
        out_type=jax.ShapeDtypeStruct((NP_PAD, C), jnp.float32),
        scratch_types=(
            [pltpu.VMEM((ppw_max * K,), jnp.int32)]
            + [pltpu.VMEM((EPC, rcols), rdt) for _ in range(RING)]
            + [pltpu.VMEM((OROWS, rcols), rdt) for _ in range(2)]
            + [pltpu.VMEM((OROWS, C), jnp.float32) for _ in range(2)]
            + [pltpu.VMEM((C,), bdt)]
            + [pltpu.SemaphoreType.DMA] * (RING + 4)
        ),
    )
    def sc_layer(t_hbm, idx_hbm, bwn_hbm, out_hbm,
                 idx_v, r0, r1, r2, r3, d0, d1, o0, o1, bwn_v,
                 g0, g1, g2, g3, ds0, ds1, os0, os1):
        cc = lax.axis_index("c")
        ss = lax.axis_index("s")
        base_pt = jnp.where(cc == 0, ss * ppw0, 16 * ppw0 + ss * ppw1)
        niter = jnp.where(cc == 0, ppw0 // OROWS, ppw1 // OROWS)
        nch = niter * CPI
        rows = (r0, r1, r2, r3)
        gsem = (g0, g1, g2, g3)
        dest = (d0, d1)
        dsem = (ds0, ds1)
        obuf = (o0, o1)
        osem = (os0, os1)

        pltpu.sync_copy(idx_hbm.at[pl.ds(base_pt * K, ppw_max * K)], idx_v)
        pltpu.sync_copy(bwn_hbm, bwn_v)
        if packed:
            bw = [bwn_v[pl.ds(gp * 32, 32)] for gp in range(GP)]
        else:
            bw = [bwn_v[pl.ds(g * 16, 16)] for g in range(G)]

        def issue(q, b):
            pltpu.async_copy(
                t_hbm.at[idx_v.at[pl.ds(q * EPC, EPC)]], rows[b], gsem[b])

        def wait(b):
            pltpu.make_async_copy(
                t_hbm.at[idx_v.at[pl.ds(0, EPC)]], rows[b], gsem[b]).wait()

        def issue_dest(it, h):
            pltpu.async_copy(
                t_hbm.at[pl.ds(base_pt + it * OROWS, OROWS)], dest[h], dsem[h])

        def wait_dest(h):
            pltpu.make_async_copy(
                t_hbm.at[pl.ds(base_pt, OROWS)], dest[h], dsem[h]).wait()

        def compute(b, j, h):
            for p in range(PTS):
                r = j * PTS + p
                if packed:
                    svp = [plsc.bitcast(
                               dest[h][r, pl.ds(C // 2 + gp * 16, 16)],
                               jnp.bfloat16) - bw[gp]
                           for gp in range(GP)]

                    def body(k, accs, p=p, svp=svp, b=b):
                        e = p * K + k
                        out = list(accs)
                        for gp in range(GP):
                            av = plsc.bitcast(
                                rows[b][e, pl.ds(C // 2 + gp * 16, 16)],
                                jnp.bfloat16)
                            fv = plsc.bitcast(
                                rows[b][e, pl.ds(gp * 16, 16)], jnp.bfloat16)
                            w = av - svp[gp]
                            w = jnp.maximum(w, 0.1 * w)
                            wlo, whi = plsc.unpack(
                                w, format=plsc.PackFormat.INTERLEAVED)
                            flo, fhi = plsc.unpack(
                                fv, format=plsc.PackFormat.INTERLEAVED)
                            out[2 * gp] = out[2 * gp] + wlo * flo
                            out[2 * gp + 1] = out[2 * gp + 1] + whi * fhi
                        return tuple(out)
                else:
                    svp = [dest[h][r, pl.ds(C + g * 16, 16)] - bw[g]
                           for g in range(G)]

                    def body(k, accs, p=p, svp=svp, b=b):
                        e = p * K + k
                        out = list(accs)
                        for g in range(G):
                            av = rows[b][e, pl.ds(C + g * 16, 16)]
                            fv = rows[b][e, pl.ds(g * 16, 16)]
                            w = av - svp[g]
                            w = jnp.maximum(w, 0.1 * w)
                            out[g] = out[g] + w * fv
                        return tuple(out)

                accs = lax.fori_loop(
                    0, K, body,
                    tuple(jnp.zeros((16,), jnp.float32) for _ in range(G)))
                for g in range(G):
                    obuf[h][r, pl.ds(g * 16, 16)] = accs[g]

        issue(0, 0)
        issue(1, 1)
        issue(2, 2)
        issue_dest(0, 0)
        issue_dest(1, 1)

        @pl.loop(0, niter // 2)
        def _it(ih):
            for h in range(2):
                it = ih * 2 + h

                @pl.when(ih > 0)
                def _(h=h):
                    pltpu.make_async_copy(
                        obuf[h], out_hbm.at[pl.ds(base_pt, OROWS)],
                        osem[h]).wait()

                wait_dest(h)
                for j in range(CPI):
                    q = it * CPI + j

                    @pl.when(q + (RING - 1) < nch)
                    def _(q=q, j=j):
                        issue(q + (RING - 1), (j + RING - 1) % RING)

                    wait(j % RING)
                    compute(j % RING, j, h)

                pltpu.async_copy(
                    obuf[h], out_hbm.at[pl.ds(base_pt + it * OROWS, OROWS)],
                    osem[h])

                @pl.when(it + 2 < niter)
                def _(it=it, h=h):
                    issue_dest(it + 2, h)

        for h in range(2):
            pltpu.make_async_copy(
                obuf[h], out_hbm.at[pl.ds(base_pt, OROWS)], osem[h]).wait()

    return sc_layer


_sc_layer1 = _make_sc_layer(128, 576, 64, packed=True)
_sc_layer2 = _make_sc_layer(64, 608, 32, packed=False)


def kernel(xyz, features, knn_indices, Wwn1, bwn1, Wlin1, blin1,
           Wwn2, bwn2, Wlin2, blin2, Wfc, bfc):
    xt = jnp.pad(features[0].T.astype(jnp.float32), ((0, NP_PAD - N), (0, 0)))
    xyzt = jnp.pad(xyz[0].T.astype(jnp.float32), ((0, NP_PAD - N), (0, 5)))
    idx = jnp.pad(knn_indices[0].astype(jnp.int32), ((0, NP_BIG - N), (0, 0)))
    idx = idx.reshape(-1)

    wlin1t = Wlin1.T
    wwn1t = jnp.pad(Wwn1.T, ((0, 5), (0, 0)))
    wwn2t = jnp.pad(Wwn2.T, ((0, 5), (0, 0)))
    wlin2t = Wlin2.T
    wfct = jnp.pad(Wfc.T, ((0, 0), (0, 5)))
    blin1_2d = blin1[None, :]
    blin2_2d = blin2[None, :]
    bfc_2d = jnp.pad(bfc, (0, 5))[None, :]

    def _pack_bias(b):
        c = b.shape[0]
        packed = b.reshape(c // 32, 2, 16).swapaxes(1, 2).reshape(c)
        return packed.astype(jnp.bfloat16)

    bwn1p = _pack_bias(bwn1)

    RB = 1024
    grid = (NP_PAD // RB,)

    def _row(c):
        return pl.BlockSpec((RB, c), lambda i: (i, 0))

    def _full(shape):
        return pl.BlockSpec(shape, lambda i: (0, 0))

    t1, a2t = pl.pallas_call(
        _prep1_body,
        grid=grid,
        in_specs=[_row(128), _row(8), _full((128, 128)), _full((1, 128)),
                  _full((8, 128)), _full((8, 64))],
        out_specs=[_row(256), _row(64)],
        out_shape=[
            jax.ShapeDtypeStruct((NP_PAD, 256), jnp.bfloat16),
            jax.ShapeDtypeStruct((NP_PAD, 64), jnp.float32),
        ],
    )(xt, xyzt, wlin1t, blin1_2d, wwn1t, wwn2t)

    t1i = lax.bitcast_convert_type(
        t1.reshape(NP_PAD, 128, 2), jnp.int32)
    out1 = _sc_layer1(t1i, idx, bwn1p)

    t2 = pl.pallas_call(
        _prep2_body,
        grid=grid,
        in_specs=[_row(128), _full((128, 64)), _full((1, 64)), _row(64)],
        out_specs=_row(128),
        out_shape=jax.ShapeDtypeStruct((NP_PAD, 128), jnp.float32),
    )(out1, wlin2t, blin2_2d, a2t)

    out2 = _sc_layer2(t2, idx, bwn2)

    r = pl.pallas_call(
        _final_body,
        grid=grid,
        in_specs=[_row(64), _full((64, 8)), _full((1, 8))],
        out_specs=_row(8),
        out_shape=jax.ShapeDtypeStruct((NP_PAD, 8), jnp.float32),
    )(out2, wfct, bfc_2d)

    return r[:N, :3].T[None]


# speedup vs baseline: 1.4279x; 1.4279x over previous
import dataclasses
import functools

import jax
import jax.numpy as jnp
from jax import lax
from jax.experimental import pallas as pl
from jax.experimental.pallas import tpu as pltpu
from jax.experimental.pallas import tpu_sc as plsc

N = 10000
K = 32
NP_PAD = 10240
NP_BIG = 11264
PTS = 4
EPC = PTS * K
CPI = 4
OROWS = CPI * PTS
RING = 4


def _lrelu(x):
    return jnp.maximum(x, 0.1 * x)


def _dot(a, b):
    return jnp.dot(a, b, preferred_element_type=jnp.float32,
                   precision=lax.Precision.HIGHEST)


def _pack_pairs(x):
    c = x.shape[1]
    u = lax.bitcast_convert_type(x, jnp.uint32)
    r = u + (jnp.uint32(0x7FFF) + ((u >> 16) & jnp.uint32(1)))
    blocks = []
    for g in range(c // 32):
        lo = r[:, 32 * g:32 * g + 16] >> 16
        hi = r[:, 32 * g + 16:32 * g + 32] & jnp.uint32(0xFFFF0000)
        blocks.append(lo | hi)
    return lax.bitcast_convert_type(jnp.concatenate(blocks, axis=1),
                                    jnp.int32)


def _prep1_body(xt_ref, xyzt_ref, wlin1t_ref, blin1_ref, wwn1t_ref,
                wwn2t_ref, t1_ref, a2_ref):
    f1 = _lrelu(_dot(xt_ref[...], wlin1t_ref[...]) + blin1_ref[...])
    a1 = _dot(xyzt_ref[...], wwn1t_ref[...])
    t1_ref[:, :64] = _pack_pairs(f1 * (1.0 / K))
    t1_ref[:, 64:] = _pack_pairs(a1)
    a2_ref[...] = _dot(xyzt_ref[...], wwn2t_ref[...])


def _prep2_body(x_ref, wlin2t_ref, blin2_ref, a2_ref, t2_ref):
    f2 = _lrelu(_dot(x_ref[...], wlin2t_ref[...]) + blin2_ref[...])
    t2_ref[:, :64] = f2 * (1.0 / K)
    t2_ref[:, 64:] = a2_ref[...]


def _final_body(x_ref, wfct_ref, bfc_ref, r_ref):
    r_ref[...] = _dot(x_ref[...], wfct_ref[...]) + bfc_ref[...]


def _make_sc_layer(C, ppw0, ppw1, packed):
    GP = C // 32
    G = C // 16
    ppw_max = max(ppw0, ppw1)
    rcols = C if packed else 2 * C
    rdt = jnp.int32 if packed else jnp.float32
    bdt = jnp.bfloat16 if packed else jnp.float32
    mesh = plsc.VectorSubcoreMesh(core_axis_name="c", subcore_axis_name="s")
    cp = pltpu.CompilerParams()
    if packed and "needs_layout_passes" in (
            pltpu.CompilerParams.__dataclass_fields__):
        cp = dataclasses.replace(cp, needs_layout_passes=False)

    @functools.partial(
        pl.kernel,
        mesh=mesh,
        compiler_params=cp,
        out_type=jax.ShapeDtypeStruct((NP_PAD, C), jnp.float32),
        scratch_types=(
            [pltpu.VMEM((ppw_max * K,), jnp.int32)]
            + [pltpu.VMEM((EPC, rcols), rdt) for _ in range(RING)]
            + [pltpu.VMEM((OROWS, rcols), rdt) for _ in range(2)]
            + [pltpu.VMEM((OROWS, C), jnp.float32) for _ in range(2)]
            + [pltpu.VMEM((C,), bdt)]
            + [pltpu.SemaphoreType.DMA] * (RING + 4)
        ),
    )
    def sc_layer(t_hbm, idx_hbm, bwn_hbm, out_hbm,
                 idx_v, r0, r1, r2, r3, d0, d1, o0, o1, bwn_v,
                 g0, g1, g2, g3, ds0, ds1, os0, os1):
        cc = lax.axis_index("c")
        ss = lax.axis_index("s")
        base_pt = jnp.where(cc == 0, ss * ppw0, 16 * ppw0 + ss * ppw1)
        niter = jnp.where(cc == 0, ppw0 // OROWS, ppw1 // OROWS)
        nch = niter * CPI
        rows = (r0, r1, r2, r3)
        gsem = (g0, g1, g2, g3)
        dest = (d0, d1)
        dsem = (ds0, ds1)
        obuf = (o0, o1)
        osem = (os0, os1)

        pltpu.sync_copy(idx_hbm.at[pl.ds(base_pt * K, ppw_max * K)], idx_v)
        pltpu.sync_copy(bwn_hbm, bwn_v)
        if packed:
            bw = [bwn_v[pl.ds(gp * 32, 32)] for gp in range(GP)]
        else:
            bw = [bwn_v[pl.ds(g * 16, 16)] for g in range(G)]

        def issue(q, b):
            pltpu.async_copy(
                t_hbm.at[idx_v.at[pl.ds(q * EPC, EPC)]], rows[b], gsem[b])

        def wait(b):
            pltpu.make_async_copy(
                t_hbm.at[idx_v.at[pl.ds(0, EPC)]], rows[b], gsem[b]).wait()

        def issue_dest(it, h):
            pltpu.async_copy(
                t_hbm.at[pl.ds(base_pt + it * OROWS, OROWS)], dest[h], dsem[h])

        def wait_dest(h):
            pltpu.make_async_copy(
                t_hbm.at[pl.ds(base_pt, OROWS)], dest[h], dsem[h]).wait()

        def compute(b, j, h):
            for p in range(PTS):
                r = j * PTS + p
                if packed:
                    svp = [plsc.bitcast(
                               dest[h][r, pl.ds(C // 2 + gp * 16, 16)],
                               jnp.bfloat16) - bw[gp]
                           for gp in range(GP)]

                    def body(k, accs, p=p, svp=svp, b=b):
                        e = p * K + k
                        out = list(accs)
                        for gp in range(GP):
                            av = plsc.bitcast(
                                rows[b][e, pl.ds(C // 2 + gp * 16, 16)],
                                jnp.bfloat16)
                            fv = plsc.bitcast(
                                rows[b][e, pl.ds(gp * 16, 16)], jnp.bfloat16)
                            w = av - svp[gp]
                            w = jnp.maximum(w, 0.1 * w)
                            wlo, whi = plsc.unpack(
                                w, format=plsc.PackFormat.INTERLEAVED)
                            flo, fhi = plsc.unpack(
                                fv, format=plsc.PackFormat.INTERLEAVED)
                            out[2 * gp] = out[2 * gp] + wlo * flo
                            out[2 * gp + 1] = out[2 * gp + 1] + whi * fhi
                        return tuple(out)
                else:
                    svp = [dest[h][r, pl.ds(C + g * 16, 16)] - bw[g]
                           for g in range(G)]

                    def body(k, accs, p=p, svp=svp, b=b):
                        e = p * K + k
                        out = list(accs)
                        for g in range(G):
                            av = rows[b][e, pl.ds(C + g * 16, 16)]
                            fv = rows[b][e, pl.ds(g * 16, 16)]
                            w = av - svp[g]
                            w = jnp.maximum(w, 0.1 * w)
                            out[g] = out[g] + w * fv
                        return tuple(out)

                accs = lax.fori_loop(
                    0, K, body,
                    tuple(jnp.zeros((16,), jnp.float32) for _ in range(G)))
                for g in range(G):
                    obuf[h][r, pl.ds(g * 16, 16)] = accs[g]

        def pipeline(ahead):
            for q in range(ahead):
                issue(q, q % RING)
            issue_dest(0, 0)
            issue_dest(1, 1)

            @pl.loop(0, niter // 2)
            def _it(ih):
                for h in range(2):
                    it = ih * 2 + h

                    @pl.when(ih > 0)
                    def _(h=h):
                        pltpu.make_async_copy(
                            obuf[h], out_hbm.at[pl.ds(base_pt, OROWS)],
                            osem[h]).wait()

                    wait_dest(h)
                    for j in range(CPI):
                        q = it * CPI + j

                        @pl.when(q + ahead < nch)
                        def _(q=q, j=j):
                            issue(q + ahead, (j + ahead) % RING)

                        wait(j % RING)
                        compute(j % RING, j, h)

                    pltpu.async_copy(
                        obuf[h],
                        out_hbm.at[pl.ds(base_pt + it * OROWS, OROWS)],
                        osem[h])

                    @pl.when(it + 2 < niter)
                    def _(it=it, h=h):
                        issue_dest(it + 2, h)

        @pl.when(cc == 0)
        def _():
            pipeline(3)

        @pl.when(cc == 1)
        def _():
            pipeline(1)

        for h in range(2):
            pltpu.make_async_copy(
                obuf[h], out_hbm.at[pl.ds(base_pt, OROWS)], osem[h]).wait()

    return sc_layer


_sc_layer1 = _make_sc_layer(128, 576, 64, packed=True)
_sc_layer2 = _make_sc_layer(64, 576, 64, packed=False)


def kernel(xyz, features, knn_indices, Wwn1, bwn1, Wlin1, blin1,
           Wwn2, bwn2, Wlin2, blin2, Wfc, bfc):
    xt = jnp.pad(features[0].T.astype(jnp.float32), ((0, NP_PAD - N), (0, 0)))
    xyzt = jnp.pad(xyz[0].T.astype(jnp.float32), ((0, NP_PAD - N), (0, 5)))
    idx = jnp.pad(knn_indices[0].astype(jnp.int32), ((0, NP_BIG - N), (0, 0)))
    idx = idx.reshape(-1)

    wlin1t = Wlin1.T
    wwn1t = jnp.pad(Wwn1.T, ((0, 5), (0, 0)))
    wwn2t = jnp.pad(Wwn2.T, ((0, 5), (0, 0)))
    wlin2t = Wlin2.T
    wfct = jnp.pad(Wfc.T, ((0, 0), (0, 5)))
    blin1_2d = blin1[None, :]
    blin2_2d = blin2[None, :]
    bfc_2d = jnp.pad(bfc, (0, 5))[None, :]

    c1 = bwn1.shape[0]
    bwn1p = bwn1.reshape(c1 // 32, 2, 16).swapaxes(1, 2).reshape(c1)
    bwn1p = bwn1p.astype(jnp.bfloat16)

    RB = 1024
    grid = (NP_PAD // RB,)

    def _row(c):
        return pl.BlockSpec((RB, c), lambda i: (i, 0))

    def _full(shape):
        return pl.BlockSpec(shape, lambda i: (0, 0))

    t1, a2t = pl.pallas_call(
        _prep1_body,
        grid=grid,
        in_specs=[_row(128), _row(8), _full((128, 128)), _full((1, 128)),
                  _full((8, 128)), _full((8, 64))],
        out_specs=[_row(128), _row(64)],
        out_shape=[
            jax.ShapeDtypeStruct((NP_PAD, 128), jnp.int32),
            jax.ShapeDtypeStruct((NP_PAD, 64), jnp.float32),
        ],
    )(xt, xyzt, wlin1t, blin1_2d, wwn1t, wwn2t)

    out1 = _sc_layer1(t1, idx, bwn1p)

    t2 = pl.pallas_call(
        _prep2_body,
        grid=grid,
        in_specs=[_row(128), _full((128, 64)), _full((1, 64)), _row(64)],
        out_specs=_row(128),
        out_shape=jax.ShapeDtypeStruct((NP_PAD, 128), jnp.float32),
    )(out1, wlin2t, blin2_2d, a2t)

    out2 = _sc_layer2(t2, idx, bwn2)

    r = pl.pallas_call(
        _final_body,
        grid=grid,
        in_specs=[_row(64), _full((64, 8)), _full((1, 8))],
        out_specs=_row(8),
        out_shape=jax.ShapeDtypeStruct((NP_PAD, 8), jnp.float32),
    )(out2, wfct, bfc_2d)

    return r[:N, :3].T[None]
